# baseline (speedup 1.0000x reference)
import functools

import jax
import jax.numpy as jnp
from jax import lax
from jax.experimental import pallas as pl
from jax.experimental.pallas import tpu as pltpu
from jax.experimental.pallas import tpu_sc as plsc

NUM_CLASSES = 1000000
LAT_DIM = 32
BATCH = 16384

_INFO = plsc.get_sparse_core_info()
_NC, _NS, _L = _INFO.num_cores, _INFO.num_subcores, _INFO.num_lanes
_NW = _NC * _NS
_NB = (NUM_CLASSES + 127) // 128
_CAP = 768
_NQ = _CAP // _L
_OUT_ROWS = BATCH + 128


def _phase1_body(idx_hbm, tabt_hbm, packed_hbm, pos_hbm,
                 idx_v, mli_v, mlp_v, pk_v, blk0_v, blk1_v, sem0, sem1):
    wid = lax.axis_index("s") * _NC + lax.axis_index("c")
    lo_b = wid * _NB // _NW
    hi_b = (wid + 1) * _NB // _NW
    lo = lo_b * 128
    hi = hi_b * 128
    lanes = lax.iota(jnp.int32, _L)

    pltpu.sync_copy(idx_hbm.at[pl.ds(0, BATCH)], idx_v)

    def init_q(q, _):
        mli_v[q >> 3, pl.ds((q & 7) * _L, _L)] = jnp.full((_L,), -1, jnp.int32)
        mlp_v[q >> 3, pl.ds((q & 7) * _L, _L)] = (
            BATCH + ((wid * _NQ + q) * _L + lanes) % 128)
        return _

    lax.fori_loop(0, _NQ, init_q, None)

    def scan_k(k, count):
        v = idx_v[pl.ds(k * _L, _L)]
        pos = k * _L + lanes
        m = (v >= lo) & (v < hi)
        pc = jnp.cumsum(jnp.where(m, 1, 0))
        slots = count + pc - 1
        m2 = m & (slots < _CAP)
        plsc.store_scatter(mli_v, [slots >> 7, slots & 127], v, mask=m2)
        plsc.store_scatter(mlp_v, [slots >> 7, slots & 127], pos, mask=m2)
        return count + pc[_L - 1]

    lax.fori_loop(0, BATCH // _L, scan_k, jnp.int32(0))

    def fire(bi, blk, sem):
        off = pl.multiple_of((lo_b + bi) * 128, 128)
        pltpu.async_copy(tabt_hbm.at[:, pl.ds(off, 128)], blk, sem)

    def process(b, blk):
        def do_q(q, _):
            mv = mli_v[q >> 3, pl.ds((q & 7) * _L, _L)]
            mb = (mv >> 7) == b
            nmatch = plsc.all_reduce_population_count(mb)

            @pl.when(nmatch[0] > 0)
            def _hit():
                cnt = jnp.cumsum(jnp.where(mb, 1, 0))

                def do_match(t, _):
                    lane = jnp.sum(jnp.where(cnt <= t, 1, 0))
                    c = jnp.sum(jnp.where(lanes == lane, mv, 0)) & 127
                    slot = q * _L + lane
                    cvec = jnp.full((_L,), c, jnp.int32)
                    svec = jnp.full((_L,), slot, jnp.int32)
                    for fb in range(2):
                        fvec = fb * _L + lanes
                        plsc.store_scatter(
                            pk_v, [fvec, svec],
                            plsc.load_gather(blk, [fvec, cvec]))
                    for fb in range(2, 4):
                        fvec = fb * _L + lanes
                        plsc.store_scatter(
                            pk_v, [fvec, svec],
                            jnp.exp(plsc.load_gather(blk, [fvec, cvec])))
                    return _

                lax.fori_loop(0, cnt[_L - 1], do_match, None)
            return _

        lax.fori_loop(0, _NQ, do_q, None)

    nb = hi_b - lo_b
    fire(0, blk0_v, sem0)

    def step(bi, _):
        @pl.when(bi % 2 == 0)
        def _even():
            pltpu.make_async_copy(
                tabt_hbm.at[:, pl.ds(0, 128)], blk0_v, sem0).wait()

            @pl.when(bi + 1 < nb)
            def _f():
                fire(bi + 1, blk1_v, sem1)
            process(lo_b + bi, blk0_v)

        @pl.when(bi % 2 == 1)
        def _odd():
            pltpu.make_async_copy(
                tabt_hbm.at[:, pl.ds(0, 128)], blk1_v, sem1).wait()

            @pl.when(bi + 1 < nb)
            def _f():
                fire(bi + 1, blk0_v, sem0)
            process(lo_b + bi, blk1_v)
        return _

    lax.fori_loop(0, nb, step, None)

    base = pl.multiple_of(wid * _CAP, 128)
    pltpu.sync_copy(pk_v, packed_hbm.at[:, pl.ds(base, _CAP)])
    pltpu.sync_copy(mlp_v, pos_hbm.at[wid])


def _phase2_body(packed_hbm, pos_hbm, out_hbm, pk_v, pos_v, rows_v, sem):
    wid = lax.axis_index("s") * _NC + lax.axis_index("c")
    base = wid * _CAP
    lanes = lax.iota(jnp.int32, _L)

    pltpu.sync_copy(packed_hbm.at[:, pl.ds(base, _CAP)], pk_v)
    pltpu.sync_copy(pos_hbm.at[wid], pos_v)

    copies = []
    for ch in range(_CAP // 128):
        def tr(g, _, ch=ch):
            s0 = ch * 128 + g * _L
            rvec = g * _L + lanes
            for f in range(64):
                plsc.store_scatter(
                    rows_v, [rvec + ch * 128, jnp.full((_L,), f, jnp.int32)],
                    pk_v[f, pl.ds(s0, _L)])
            return _

        lax.fori_loop(0, 128 // _L, tr, None)
        copies.append(pltpu.async_copy(
            rows_v.at[pl.ds(ch * 128, 128), :],
            out_hbm.at[pos_v.at[ch]], sem))
    for c in copies:
        c.wait()


@jax.jit
def _prior_sc(indices, table_t):
    f32, i32 = jnp.float32, jnp.int32
    phase1 = functools.partial(
        pl.kernel,
        out_type=(jax.ShapeDtypeStruct((64, _NW * _CAP), f32),
                  jax.ShapeDtypeStruct((_NW, _CAP // 128, 128), i32)),
        mesh=plsc.VectorSubcoreMesh(core_axis_name="c", subcore_axis_name="s"),
        compiler_params=pltpu.CompilerParams(needs_layout_passes=False),
        scratch_types=[
            pltpu.VMEM((BATCH,), i32),
            pltpu.VMEM((_NQ // 8, 8 * _L), i32),
            pltpu.VMEM((_CAP // 128, 128), i32),
            pltpu.VMEM((64, _CAP), f32),
            pltpu.VMEM((64, 128), f32),
            pltpu.VMEM((64, 128), f32),
            pltpu.SemaphoreType.DMA,
            pltpu.SemaphoreType.DMA,
        ],
    )(_phase1_body)
    packed, poslist = phase1(indices, table_t)

    phase2 = functools.partial(
        pl.kernel,
        out_type=jax.ShapeDtypeStruct((_OUT_ROWS, 64), f32),
        mesh=plsc.VectorSubcoreMesh(core_axis_name="c", subcore_axis_name="s"),
        compiler_params=pltpu.CompilerParams(use_tc_tiling_on_sc=False,
                                             needs_layout_passes=False),
        scratch_types=[
            pltpu.VMEM((64, _CAP), f32),
            pltpu.VMEM((_CAP // 128, 128), i32),
            pltpu.VMEM((_CAP, 64), f32),
            pltpu.SemaphoreType.DMA,
        ],
    )(_phase2_body)
    return phase2(packed, poslist)


def kernel(indices, table):
    out = _prior_sc(indices.astype(jnp.int32), table.T)
    return (out[:BATCH, :LAT_DIM], out[:BATCH, LAT_DIM:])


# speedup vs baseline: 1.3080x; 1.3080x over previous
import functools

import jax
import jax.numpy as jnp
from jax import lax
from jax.experimental import pallas as pl
from jax.experimental.pallas import tpu as pltpu
from jax.experimental.pallas import tpu_sc as plsc

NUM_CLASSES = 1000000
LAT_DIM = 32
BATCH = 16384

_INFO = plsc.get_sparse_core_info()
_NC, _NS, _L = _INFO.num_cores, _INFO.num_subcores, _INFO.num_lanes
_NW = _NC * _NS
_NB = (NUM_CLASSES + 127) // 128
_CAP = 768
_NQ = _CAP // _L
_OUT_ROWS = BATCH + 128


def _phase1_body(idx_hbm, tabt_hbm, packed_hbm, pos_hbm,
                 idx_v, mli_v, mlp_v, pk_v, blk0_v, blk1_v, sem0, sem1):
    wid = lax.axis_index("s") * _NC + lax.axis_index("c")
    lo_b = wid * _NB // _NW
    hi_b = (wid + 1) * _NB // _NW
    lo = lo_b * 128
    hi = hi_b * 128
    lanes = lax.iota(jnp.int32, _L)

    pltpu.sync_copy(idx_hbm.at[pl.ds(0, BATCH)], idx_v)

    def init_q(q, _):
        mli_v[q >> 3, pl.ds((q & 7) * _L, _L)] = jnp.full((_L,), -1, jnp.int32)
        mlp_v[q >> 3, pl.ds((q & 7) * _L, _L)] = (
            BATCH + ((wid * _NQ + q) * _L + lanes) % 128)
        return _

    lax.fori_loop(0, _NQ, init_q, None)

    def scan_k(k, count):
        v = idx_v[pl.ds(k * _L, _L)]
        pos = k * _L + lanes
        m = (v >= lo) & (v < hi)
        pc = jnp.cumsum(jnp.where(m, 1, 0))
        slots = count + pc - 1
        m2 = m & (slots < _CAP)
        plsc.store_scatter(mli_v, [slots >> 7, slots & 127], v, mask=m2)
        plsc.store_scatter(mlp_v, [slots >> 7, slots & 127], pos, mask=m2)
        return count + pc[_L - 1]

    lax.fori_loop(0, BATCH // _L, scan_k, jnp.int32(0))

    def fire(bi, blk, sem):
        off = pl.multiple_of((lo_b + bi) * 128, 128)
        pltpu.async_copy(tabt_hbm.at[:, pl.ds(off, 128)], blk, sem)

    def process(b, blk):
        def do_q(q, _):
            mv = mli_v[q >> 3, pl.ds((q & 7) * _L, _L)]
            mb = (mv >> 7) == b
            cnt = jnp.cumsum(jnp.where(mb, 1, 0))

            def do_match(t, _):
                lane = jnp.sum(jnp.where(cnt <= t, 1, 0))
                c = jnp.sum(jnp.where(lanes == lane, mv, 0)) & 127
                slot = q * _L + lane
                cvec = jnp.full((_L,), c, jnp.int32)
                svec = jnp.full((_L,), slot, jnp.int32)
                for fb in range(2):
                    fvec = fb * _L + lanes
                    plsc.store_scatter(
                        pk_v, [fvec, svec],
                        plsc.load_gather(blk, [fvec, cvec]))
                for fb in range(2, 4):
                    fvec = fb * _L + lanes
                    plsc.store_scatter(
                        pk_v, [fvec, svec],
                        jnp.exp(plsc.load_gather(blk, [fvec, cvec])))
                return _

            lax.fori_loop(0, cnt[_L - 1], do_match, None)
            return _

        lax.fori_loop(0, _NQ, do_q, None)

    nb = hi_b - lo_b
    fire(0, blk0_v, sem0)

    def step(bi, _):
        @pl.when(bi % 2 == 0)
        def _even():
            pltpu.make_async_copy(
                tabt_hbm.at[:, pl.ds(0, 128)], blk0_v, sem0).wait()

            @pl.when(bi + 1 < nb)
            def _f():
                fire(bi + 1, blk1_v, sem1)
            process(lo_b + bi, blk0_v)

        @pl.when(bi % 2 == 1)
        def _odd():
            pltpu.make_async_copy(
                tabt_hbm.at[:, pl.ds(0, 128)], blk1_v, sem1).wait()

            @pl.when(bi + 1 < nb)
            def _f():
                fire(bi + 1, blk0_v, sem0)
            process(lo_b + bi, blk1_v)
        return _

    lax.fori_loop(0, nb, step, None)

    base = pl.multiple_of(wid * _CAP, 128)
    pltpu.sync_copy(pk_v, packed_hbm.at[:, pl.ds(base, _CAP)])
    pltpu.sync_copy(mlp_v, pos_hbm.at[wid])


def _phase2_body(packed_hbm, pos_hbm, out_hbm, pk_v, pos_v, rows_v, sem):
    wid = lax.axis_index("s") * _NC + lax.axis_index("c")
    base = wid * _CAP
    lanes = lax.iota(jnp.int32, _L)

    pltpu.sync_copy(packed_hbm.at[:, pl.ds(base, _CAP)], pk_v)
    pltpu.sync_copy(pos_hbm.at[wid], pos_v)

    copies = []
    for ch in range(_CAP // 128):
        def tr(g, _, ch=ch):
            s0 = ch * 128 + g * _L
            rvec = g * _L + lanes
            for f in range(64):
                plsc.store_scatter(
                    rows_v, [rvec + ch * 128, jnp.full((_L,), f, jnp.int32)],
                    pk_v[f, pl.ds(s0, _L)])
            return _

        lax.fori_loop(0, 128 // _L, tr, None)
        copies.append(pltpu.async_copy(
            rows_v.at[pl.ds(ch * 128, 128), :],
            out_hbm.at[pos_v.at[ch]], sem))
    for c in copies:
        c.wait()


@jax.jit
def _prior_sc(indices, table_t):
    f32, i32 = jnp.float32, jnp.int32
    phase1 = functools.partial(
        pl.kernel,
        out_type=(jax.ShapeDtypeStruct((64, _NW * _CAP), f32),
                  jax.ShapeDtypeStruct((_NW, _CAP // 128, 128), i32)),
        mesh=plsc.VectorSubcoreMesh(core_axis_name="c", subcore_axis_name="s"),
        compiler_params=pltpu.CompilerParams(needs_layout_passes=False),
        scratch_types=[
            pltpu.VMEM((BATCH,), i32),
            pltpu.VMEM((_NQ // 8, 8 * _L), i32),
            pltpu.VMEM((_CAP // 128, 128), i32),
            pltpu.VMEM((64, _CAP), f32),
            pltpu.VMEM((64, 128), f32),
            pltpu.VMEM((64, 128), f32),
            pltpu.SemaphoreType.DMA,
            pltpu.SemaphoreType.DMA,
        ],
    )(_phase1_body)
    packed, poslist = phase1(indices, table_t)

    phase2 = functools.partial(
        pl.kernel,
        out_type=jax.ShapeDtypeStruct((_OUT_ROWS, 64), f32),
        mesh=plsc.VectorSubcoreMesh(core_axis_name="c", subcore_axis_name="s"),
        compiler_params=pltpu.CompilerParams(use_tc_tiling_on_sc=False,
                                             needs_layout_passes=False),
        scratch_types=[
            pltpu.VMEM((64, _CAP), f32),
            pltpu.VMEM((_CAP // 128, 128), i32),
            pltpu.VMEM((_CAP, 64), f32),
            pltpu.SemaphoreType.DMA,
        ],
    )(_phase2_body)
    return phase2(packed, poslist)


def kernel(indices, table):
    out = _prior_sc(indices.astype(jnp.int32), table.T)
    return (out[:BATCH, :LAT_DIM], out[:BATCH, LAT_DIM:])


# speedup vs baseline: 1.5351x; 1.1736x over previous
import functools

import jax
import jax.numpy as jnp
from jax import lax
from jax.experimental import pallas as pl
from jax.experimental.pallas import tpu as pltpu
from jax.experimental.pallas import tpu_sc as plsc

NUM_CLASSES = 1000000
LAT_DIM = 32
BATCH = 16384

_INFO = plsc.get_sparse_core_info()
_NC, _NS, _L = _INFO.num_cores, _INFO.num_subcores, _INFO.num_lanes
_NW = _NC * _NS
_BPW = BATCH // _NW
_CHUNK = 64
_NCHUNK = _BPW // _CHUNK
_GRP = 8


def _body(idx_hbm, tab_hbm, mut_hbm, sigt_hbm,
          idx_v, rows_v, mut_v, sigt_v, sem):
    wid = lax.axis_index("s") * _NC + lax.axis_index("c")
    base = wid * _BPW

    pltpu.sync_copy(idx_hbm.at[pl.ds(base, _BPW)], idx_v)

    lanes = lax.iota(jnp.int32, _L)

    def do_chunk(j, _):
        def fire(i16, _):
            v = idx_v[pl.ds(j * _CHUNK + i16 * _L, _L)]
            for s in range(_L):
                r8 = pl.multiple_of((v[s] >> 3) * _GRP, _GRP)
                pltpu.async_copy(
                    tab_hbm.at[pl.ds(r8, _GRP), :],
                    rows_v.at[pl.ds((i16 * _L + s) * _GRP, _GRP), :], sem)
            return _

        lax.fori_loop(0, _CHUNK // _L, fire, None)

        def drain(i, _):
            pltpu.make_async_copy(
                tab_hbm.at[pl.ds(0, _GRP), :],
                rows_v.at[pl.ds(0, _GRP), :], sem).wait()
            return _

        lax.fori_loop(0, _CHUNK, drain, None)

        def select(g, _):
            o = j * _CHUNK + g * _L
            loc = idx_v[pl.ds(o, _L)] & 7
            row16 = (g * _L + lanes) * _GRP + loc
            for f in range(LAT_DIM):
                mut_v[f, pl.ds(o, _L)] = plsc.load_gather(
                    rows_v, [row16, jnp.full((_L,), f, jnp.int32)])
            for f in range(LAT_DIM):
                sigt_v[f, pl.ds(o, _L)] = jnp.exp(plsc.load_gather(
                    rows_v, [row16, jnp.full((_L,), LAT_DIM + f, jnp.int32)]))
            return _

        lax.fori_loop(0, _CHUNK // _L, select, None)
        return _

    lax.fori_loop(0, _NCHUNK, do_chunk, None)

    pltpu.sync_copy(mut_v, mut_hbm.at[:, pl.ds(base, _BPW)])
    pltpu.sync_copy(sigt_v, sigt_hbm.at[:, pl.ds(base, _BPW)])


@jax.jit
def _prior_sc(indices, table):
    f32 = jnp.float32
    run = functools.partial(
        pl.kernel,
        out_type=(jax.ShapeDtypeStruct((LAT_DIM, BATCH), f32),
                  jax.ShapeDtypeStruct((LAT_DIM, BATCH), f32)),
        mesh=plsc.VectorSubcoreMesh(core_axis_name="c", subcore_axis_name="s"),
        compiler_params=pltpu.CompilerParams(needs_layout_passes=False),
        scratch_types=[
            pltpu.VMEM((_BPW,), jnp.int32),
            pltpu.VMEM((_CHUNK * _GRP, 64), f32),
            pltpu.VMEM((LAT_DIM, _BPW), f32),
            pltpu.VMEM((LAT_DIM, _BPW), f32),
            pltpu.SemaphoreType.DMA,
        ],
    )(_body)
    return run(indices, table)


def kernel(indices, table):
    mu_t, sigma_t = _prior_sc(indices.astype(jnp.int32), table)
    return (mu_t.T, sigma_t.T)


# speedup vs baseline: 1.6973x; 1.1057x over previous
import functools

import jax
import jax.numpy as jnp
from jax import lax
from jax.experimental import pallas as pl
from jax.experimental.pallas import tpu as pltpu
from jax.experimental.pallas import tpu_sc as plsc

NUM_CLASSES = 1000000
LAT_DIM = 32
BATCH = 16384

_INFO = plsc.get_sparse_core_info()
_NC, _NS, _L = _INFO.num_cores, _INFO.num_subcores, _INFO.num_lanes
_NW = _NC * _NS
_NB = (NUM_CLASSES + 127) // 128
_CAP = 768
_NQ = _CAP // _L
_OUT_ROWS = BATCH + 128


_NG = 16
_GCAP = 96
_GQ = _GCAP // _L


def _phase1_body(idx_hbm, tabt_hbm, packed_hbm, pos_hbm,
                 idx_v, mli_v, mlp_v, slc_v, sls_v, pk_v,
                 blk0_v, blk1_v, sem0, sem1):
    wid = lax.axis_index("s") * _NC + lax.axis_index("c")
    lo_b = wid * _NB // _NW
    hi_b = (wid + 1) * _NB // _NW
    lo = lo_b * 128
    hi = hi_b * 128
    lanes = lax.iota(jnp.int32, _L)

    pltpu.sync_copy(idx_hbm.at[pl.ds(0, BATCH)], idx_v)

    def init_q(q, _):
        mli_v[q >> 3, pl.ds((q & 7) * _L, _L)] = jnp.full((_L,), -1, jnp.int32)
        mlp_v[q >> 3, pl.ds((q & 7) * _L, _L)] = (
            BATCH + ((wid * _NQ + q) * _L + lanes) % 128)
        return _

    lax.fori_loop(0, _NQ, init_q, None)

    def scan_k(k, count):
        v = idx_v[pl.ds(k * _L, _L)]
        pos = k * _L + lanes
        m = (v >= lo) & (v < hi)
        pc = jnp.cumsum(jnp.where(m, 1, 0))
        slots = count + pc - 1
        m2 = m & (slots < _CAP)
        plsc.store_scatter(mli_v, [slots >> 7, slots & 127], v, mask=m2)
        plsc.store_scatter(mlp_v, [slots >> 7, slots & 127], pos, mask=m2)
        return count + pc[_L - 1]

    lax.fori_loop(0, BATCH // _L, scan_k, jnp.int32(0))

    def init_sub(t, _):
        f0 = t * _L
        slc_v[f0 >> 7, pl.ds(f0 & 127, _L)] = jnp.full((_L,), -1, jnp.int32)
        return _

    lax.fori_loop(0, _NG * _GQ, init_sub, None)

    def bucket_g(g, _):
        glo = lo_b + g * (_NB // _NW // _NG + 1)
        ghi = glo + (_NB // _NW // _NG + 1)

        def bucket_q(q, gcount):
            mv = mli_v[q >> 3, pl.ds((q & 7) * _L, _L)]
            bid = mv >> 7
            m = (bid >= glo) & (bid < ghi)
            pc = jnp.cumsum(jnp.where(m, 1, 0))
            gslots = g * _GCAP + gcount + pc - 1
            m2 = m & (gcount + pc - 1 < _GCAP)
            plsc.store_scatter(slc_v, [gslots >> 7, gslots & 127], mv, mask=m2)
            plsc.store_scatter(sls_v, [gslots >> 7, gslots & 127],
                               q * _L + lanes, mask=m2)
            return gcount + pc[_L - 1]

        lax.fori_loop(0, _NQ, bucket_q, jnp.int32(0))
        return _

    lax.fori_loop(0, _NG, bucket_g, None)

    def fire(bi, blk, sem):
        off = pl.multiple_of((lo_b + bi) * 128, 128)
        pltpu.async_copy(tabt_hbm.at[:, pl.ds(off, 128)], blk, sem)

    def process(b, blk):
        g = (b - lo_b) // (_NB // _NW // _NG + 1)

        def do_q(t, _):
            f0 = (g * _GQ + t) * _L
            mv = slc_v[f0 >> 7, pl.ds(f0 & 127, _L)]
            sv = sls_v[f0 >> 7, pl.ds(f0 & 127, _L)]
            mb = (mv >> 7) == b
            cnt = jnp.cumsum(jnp.where(mb, 1, 0))

            def do_match(tt, _):
                lane = jnp.sum(jnp.where(cnt <= tt, 1, 0))
                onlane = lanes == lane
                c = jnp.sum(jnp.where(onlane, mv, 0)) & 127
                slot = jnp.sum(jnp.where(onlane, sv, 0))
                cvec = jnp.full((_L,), c, jnp.int32)
                svec = jnp.full((_L,), slot, jnp.int32)
                for fb in range(2):
                    fvec = fb * _L + lanes
                    plsc.store_scatter(
                        pk_v, [fvec, svec],
                        plsc.load_gather(blk, [fvec, cvec]))
                for fb in range(2, 4):
                    fvec = fb * _L + lanes
                    plsc.store_scatter(
                        pk_v, [fvec, svec],
                        jnp.exp(plsc.load_gather(blk, [fvec, cvec])))
                return _

            lax.fori_loop(0, cnt[_L - 1], do_match, None)
            return _

        lax.fori_loop(0, _GQ, do_q, None)

    nb = hi_b - lo_b
    fire(0, blk0_v, sem0)

    def step(bi, _):
        @pl.when(bi % 2 == 0)
        def _even():
            pltpu.make_async_copy(
                tabt_hbm.at[:, pl.ds(0, 128)], blk0_v, sem0).wait()

            @pl.when(bi + 1 < nb)
            def _f():
                fire(bi + 1, blk1_v, sem1)
            process(lo_b + bi, blk0_v)

        @pl.when(bi % 2 == 1)
        def _odd():
            pltpu.make_async_copy(
                tabt_hbm.at[:, pl.ds(0, 128)], blk1_v, sem1).wait()

            @pl.when(bi + 1 < nb)
            def _f():
                fire(bi + 1, blk0_v, sem0)
            process(lo_b + bi, blk1_v)
        return _

    lax.fori_loop(0, nb, step, None)

    base = pl.multiple_of(wid * _CAP, 128)
    pltpu.sync_copy(pk_v, packed_hbm.at[:, pl.ds(base, _CAP)])
    pltpu.sync_copy(mlp_v, pos_hbm.at[wid])


def _phase2_body(packed_hbm, pos_hbm, out_hbm, pk_v, pos_v, rows_v, sem):
    wid = lax.axis_index("s") * _NC + lax.axis_index("c")
    base = wid * _CAP
    lanes = lax.iota(jnp.int32, _L)

    pltpu.sync_copy(packed_hbm.at[:, pl.ds(base, _CAP)], pk_v)
    pltpu.sync_copy(pos_hbm.at[wid], pos_v)

    copies = []
    for ch in range(_CAP // 128):
        def tr(g, _, ch=ch):
            s0 = ch * 128 + g * _L
            rvec = g * _L + lanes
            for f in range(64):
                plsc.store_scatter(
                    rows_v, [rvec + ch * 128, jnp.full((_L,), f, jnp.int32)],
                    pk_v[f, pl.ds(s0, _L)])
            return _

        lax.fori_loop(0, 128 // _L, tr, None)
        copies.append(pltpu.async_copy(
            rows_v.at[pl.ds(ch * 128, 128), :],
            out_hbm.at[pos_v.at[ch]], sem))
    for c in copies:
        c.wait()


@jax.jit
def _prior_sc(indices, table_t):
    f32, i32 = jnp.float32, jnp.int32
    phase1 = functools.partial(
        pl.kernel,
        out_type=(jax.ShapeDtypeStruct((64, _NW * _CAP), f32),
                  jax.ShapeDtypeStruct((_NW, _CAP // 128, 128), i32)),
        mesh=plsc.VectorSubcoreMesh(core_axis_name="c", subcore_axis_name="s"),
        compiler_params=pltpu.CompilerParams(needs_layout_passes=False),
        scratch_types=[
            pltpu.VMEM((BATCH,), i32),
            pltpu.VMEM((_NQ // 8, 8 * _L), i32),
            pltpu.VMEM((_CAP // 128, 128), i32),
            pltpu.VMEM((_NG * _GCAP // 128, 128), i32),
            pltpu.VMEM((_NG * _GCAP // 128, 128), i32),
            pltpu.VMEM((64, _CAP), f32),
            pltpu.VMEM((64, 128), f32),
            pltpu.VMEM((64, 128), f32),
            pltpu.SemaphoreType.DMA,
            pltpu.SemaphoreType.DMA,
        ],
    )(_phase1_body)
    packed, poslist = phase1(indices, table_t)

    phase2 = functools.partial(
        pl.kernel,
        out_type=jax.ShapeDtypeStruct((_OUT_ROWS, 64), f32),
        mesh=plsc.VectorSubcoreMesh(core_axis_name="c", subcore_axis_name="s"),
        compiler_params=pltpu.CompilerParams(use_tc_tiling_on_sc=False,
                                             needs_layout_passes=False),
        scratch_types=[
            pltpu.VMEM((64, _CAP), f32),
            pltpu.VMEM((_CAP // 128, 128), i32),
            pltpu.VMEM((_CAP, 64), f32),
            pltpu.SemaphoreType.DMA,
        ],
    )(_phase2_body)
    return phase2(packed, poslist)


def kernel(indices, table):
    out = _prior_sc(indices.astype(jnp.int32), table.T)
    return (out[:BATCH, :LAT_DIM], out[:BATCH, LAT_DIM:])


# speedup vs baseline: 2.4054x; 1.4172x over previous
import functools

import jax
import jax.numpy as jnp
from jax import lax
from jax.experimental import pallas as pl
from jax.experimental.pallas import tpu as pltpu
from jax.experimental.pallas import tpu_sc as plsc

NUM_CLASSES = 1000000
LAT_DIM = 32
BATCH = 16384

_INFO = plsc.get_sparse_core_info()
_NC, _NS, _L = _INFO.num_cores, _INFO.num_subcores, _INFO.num_lanes
_NW = _NC * _NS
_NB = (NUM_CLASSES + 127) // 128
_CAP = 768
_NQ = _CAP // _L
_OUT_ROWS = BATCH + 128


_NG = 16
_GCAP = 96
_GQ = _GCAP // _L


def _phase1_body(idx_hbm, tabt_hbm, packed_hbm, pos_hbm,
                 idx_v, mli_v, mlp_v, slc_v, sls_v, pk_v,
                 blk0_v, blk1_v, blk2_v, sem0, sem1, sem2):
    wid = lax.axis_index("s") * _NC + lax.axis_index("c")
    lo_b = wid * _NB // _NW
    hi_b = (wid + 1) * _NB // _NW
    lo = lo_b * 128
    hi = hi_b * 128
    lanes = lax.iota(jnp.int32, _L)

    pltpu.sync_copy(idx_hbm.at[pl.ds(0, BATCH)], idx_v)

    def init_q(q, _):
        mli_v[q >> 3, pl.ds((q & 7) * _L, _L)] = jnp.full((_L,), -1, jnp.int32)
        mlp_v[q >> 3, pl.ds((q & 7) * _L, _L)] = (
            BATCH + ((wid * _NQ + q) * _L + lanes) % 128)
        return _

    lax.fori_loop(0, _NQ, init_q, None)

    def scan_k(k, count):
        v = idx_v[pl.ds(k * _L, _L)]
        pos = k * _L + lanes
        m = (v >= lo) & (v < hi)
        pc = jnp.cumsum(jnp.where(m, 1, 0))
        slots = count + pc - 1
        m2 = m & (slots < _CAP)
        plsc.store_scatter(mli_v, [slots >> 7, slots & 127], v, mask=m2)
        plsc.store_scatter(mlp_v, [slots >> 7, slots & 127], pos, mask=m2)
        return count + pc[_L - 1]

    lax.fori_loop(0, BATCH // _L, scan_k, jnp.int32(0))

    def init_sub(t, _):
        f0 = t * _L
        slc_v[f0 >> 7, pl.ds(f0 & 127, _L)] = jnp.full((_L,), -1, jnp.int32)
        return _

    lax.fori_loop(0, _NG * _GQ, init_sub, None)

    def bucket_g(g, _):
        glo = lo_b + g * (_NB // _NW // _NG + 1)
        ghi = glo + (_NB // _NW // _NG + 1)

        def bucket_q(q, gcount):
            mv = mli_v[q >> 3, pl.ds((q & 7) * _L, _L)]
            bid = mv >> 7
            m = (bid >= glo) & (bid < ghi)
            pc = jnp.cumsum(jnp.where(m, 1, 0))
            gslots = g * _GCAP + gcount + pc - 1
            m2 = m & (gcount + pc - 1 < _GCAP)
            plsc.store_scatter(slc_v, [gslots >> 7, gslots & 127], mv, mask=m2)
            plsc.store_scatter(sls_v, [gslots >> 7, gslots & 127],
                               q * _L + lanes, mask=m2)
            return gcount + pc[_L - 1]

        lax.fori_loop(0, _NQ, bucket_q, jnp.int32(0))
        return _

    lax.fori_loop(0, _NG, bucket_g, None)

    def fire(bi, blk, sem):
        off = pl.multiple_of((lo_b + bi) * 128, 128)
        pltpu.async_copy(tabt_hbm.at[:, pl.ds(off, 128)], blk, sem)

    def process(b, blk):
        g = (b - lo_b) // (_NB // _NW // _NG + 1)

        def do_q(t, _):
            f0 = (g * _GQ + t) * _L
            mv = slc_v[f0 >> 7, pl.ds(f0 & 127, _L)]
            sv = sls_v[f0 >> 7, pl.ds(f0 & 127, _L)]
            mb = (mv >> 7) == b
            cnt = jnp.cumsum(jnp.where(mb, 1, 0))

            def do_match(tt, _):
                lane = jnp.sum(jnp.where(cnt <= tt, 1, 0))
                onlane = lanes == lane
                c = jnp.sum(jnp.where(onlane, mv, 0)) & 127
                slot = jnp.sum(jnp.where(onlane, sv, 0))
                cvec = jnp.full((_L,), c, jnp.int32)
                svec = jnp.full((_L,), slot, jnp.int32)
                for fb in range(2):
                    fvec = fb * _L + lanes
                    plsc.store_scatter(
                        pk_v, [fvec, svec],
                        plsc.load_gather(blk, [fvec, cvec]))
                for fb in range(2, 4):
                    fvec = fb * _L + lanes
                    plsc.store_scatter(
                        pk_v, [fvec, svec],
                        jnp.exp(plsc.load_gather(blk, [fvec, cvec])))
                return _

            lax.fori_loop(0, cnt[_L - 1], do_match, None)
            return _

        lax.fori_loop(0, _GQ, do_q, None)

    nb = hi_b - lo_b
    bufs = ((blk0_v, sem0), (blk1_v, sem1), (blk2_v, sem2))
    fire(0, blk0_v, sem0)
    fire(1, blk1_v, sem1)
    fire(2, blk2_v, sem2)

    def step(bi, _):
        for k in range(3):
            @pl.when(bi % 3 == k)
            def _k(k=k):
                blk, sem = bufs[k]
                nblk, nsem = bufs[k]
                pltpu.make_async_copy(
                    tabt_hbm.at[:, pl.ds(0, 128)], blk, sem).wait()
                process(lo_b + bi, blk)

                @pl.when(bi + 3 < nb)
                def _f():
                    fire(bi + 3, nblk, nsem)
        return _

    lax.fori_loop(0, nb, step, None)

    base = pl.multiple_of(wid * _CAP, 128)
    pltpu.sync_copy(pk_v, packed_hbm.at[:, pl.ds(base, _CAP)])
    pltpu.sync_copy(mlp_v, pos_hbm.at[wid])


def _phase2_body(packed_hbm, pos_hbm, out_hbm, pk_v, pos_v, rows_v, sem):
    wid = lax.axis_index("s") * _NC + lax.axis_index("c")
    base = wid * _CAP
    lanes = lax.iota(jnp.int32, _L)

    pltpu.sync_copy(packed_hbm.at[:, pl.ds(base, _CAP)], pk_v)
    pltpu.sync_copy(pos_hbm.at[wid], pos_v)

    copies = []
    for ch in range(_CAP // 128):
        def tr(g, _, ch=ch):
            s0 = ch * 128 + g * _L
            rvec = g * _L + lanes
            for f in range(64):
                plsc.store_scatter(
                    rows_v, [rvec + ch * 128, jnp.full((_L,), f, jnp.int32)],
                    pk_v[f, pl.ds(s0, _L)])
            return _

        lax.fori_loop(0, 128 // _L, tr, None)
        copies.append(pltpu.async_copy(
            rows_v.at[pl.ds(ch * 128, 128), :],
            out_hbm.at[pos_v.at[ch]], sem))
    for c in copies:
        c.wait()


@jax.jit
def _prior_sc(indices, table_t):
    f32, i32 = jnp.float32, jnp.int32
    phase1 = functools.partial(
        pl.kernel,
        out_type=(jax.ShapeDtypeStruct((64, _NW * _CAP), f32),
                  jax.ShapeDtypeStruct((_NW, _CAP // 128, 128), i32)),
        mesh=plsc.VectorSubcoreMesh(core_axis_name="c", subcore_axis_name="s"),
        compiler_params=pltpu.CompilerParams(needs_layout_passes=False),
        scratch_types=[
            pltpu.VMEM((BATCH,), i32),
            pltpu.VMEM((_NQ // 8, 8 * _L), i32),
            pltpu.VMEM((_CAP // 128, 128), i32),
            pltpu.VMEM((_NG * _GCAP // 128, 128), i32),
            pltpu.VMEM((_NG * _GCAP // 128, 128), i32),
            pltpu.VMEM((64, _CAP), f32),
            pltpu.VMEM((64, 128), f32),
            pltpu.VMEM((64, 128), f32),
            pltpu.VMEM((64, 128), f32),
            pltpu.SemaphoreType.DMA,
            pltpu.SemaphoreType.DMA,
            pltpu.SemaphoreType.DMA,
        ],
    )(_phase1_body)
    packed, poslist = phase1(indices, table_t)

    phase2 = functools.partial(
        pl.kernel,
        out_type=jax.ShapeDtypeStruct((_OUT_ROWS, 64), f32),
        mesh=plsc.VectorSubcoreMesh(core_axis_name="c", subcore_axis_name="s"),
        compiler_params=pltpu.CompilerParams(use_tc_tiling_on_sc=False,
                                             needs_layout_passes=False),
        scratch_types=[
            pltpu.VMEM((64, _CAP), f32),
            pltpu.VMEM((_CAP // 128, 128), i32),
            pltpu.VMEM((_CAP, 64), f32),
            pltpu.SemaphoreType.DMA,
        ],
    )(_phase2_body)
    return phase2(packed, poslist)


def kernel(indices, table):
    out = _prior_sc(indices.astype(jnp.int32), table.T)
    return (out[:BATCH, :LAT_DIM], out[:BATCH, LAT_DIM:])


# speedup vs baseline: 2.6613x; 1.1064x over previous
import functools

import jax
import jax.numpy as jnp
from jax import lax
from jax.experimental import pallas as pl
from jax.experimental.pallas import tpu as pltpu
from jax.experimental.pallas import tpu_sc as plsc

NUM_CLASSES = 1000000
LAT_DIM = 32
BATCH = 16384

_INFO = plsc.get_sparse_core_info()
_NC, _NS, _L = _INFO.num_cores, _INFO.num_subcores, _INFO.num_lanes
_NW = _NC * _NS
_NB = (NUM_CLASSES + 127) // 128
_CAP = 768
_NQ = _CAP // _L
_OUT_ROWS = BATCH + 128


_NG = 16
_GCAP = 96
_GQ = _GCAP // _L


def _phase1_body(idx_hbm, tabt_hbm, packed_hbm, pos_hbm,
                 idx_v, mli_v, mlp_v, slc_v, sls_v, pk_v,
                 blk0_v, blk1_v, blk2_v, blk3_v, blk4_v,
                 sem0, sem1, sem2, sem3, sem4):
    wid = lax.axis_index("s") * _NC + lax.axis_index("c")
    lo_b = wid * _NB // _NW
    hi_b = (wid + 1) * _NB // _NW
    lo = lo_b * 128
    hi = hi_b * 128
    lanes = lax.iota(jnp.int32, _L)

    pltpu.sync_copy(idx_hbm.at[pl.ds(0, BATCH)], idx_v)

    def init_q(q, _):
        mli_v[q >> 3, pl.ds((q & 7) * _L, _L)] = jnp.full((_L,), -1, jnp.int32)
        mlp_v[q >> 3, pl.ds((q & 7) * _L, _L)] = (
            BATCH + ((wid * _NQ + q) * _L + lanes) % 128)
        return _

    lax.fori_loop(0, _NQ, init_q, None)

    def scan_k(k, count):
        v = idx_v[pl.ds(k * _L, _L)]
        pos = k * _L + lanes
        m = (v >= lo) & (v < hi)
        pc = jnp.cumsum(jnp.where(m, 1, 0))
        slots = count + pc - 1
        m2 = m & (slots < _CAP)
        plsc.store_scatter(mli_v, [slots >> 7, slots & 127], v, mask=m2)
        plsc.store_scatter(mlp_v, [slots >> 7, slots & 127], pos, mask=m2)
        return count + pc[_L - 1]

    lax.fori_loop(0, BATCH // _L, scan_k, jnp.int32(0))

    def init_sub(t, _):
        f0 = t * _L
        slc_v[f0 >> 7, pl.ds(f0 & 127, _L)] = jnp.full((_L,), -1, jnp.int32)
        return _

    lax.fori_loop(0, _NG * _GQ, init_sub, None)

    def bucket_g(g, _):
        glo = lo_b + g * (_NB // _NW // _NG + 1)
        ghi = glo + (_NB // _NW // _NG + 1)

        def bucket_q(q, gcount):
            mv = mli_v[q >> 3, pl.ds((q & 7) * _L, _L)]
            bid = mv >> 7
            m = (bid >= glo) & (bid < ghi)
            pc = jnp.cumsum(jnp.where(m, 1, 0))
            gslots = g * _GCAP + gcount + pc - 1
            m2 = m & (gcount + pc - 1 < _GCAP)
            plsc.store_scatter(slc_v, [gslots >> 7, gslots & 127], mv, mask=m2)
            plsc.store_scatter(sls_v, [gslots >> 7, gslots & 127],
                               q * _L + lanes, mask=m2)
            return gcount + pc[_L - 1]

        lax.fori_loop(0, _NQ, bucket_q, jnp.int32(0))
        return _

    lax.fori_loop(0, _NG, bucket_g, None)

    def fire(bi, blk, sem):
        off = pl.multiple_of((lo_b + bi) * 128, 128)
        pltpu.async_copy(tabt_hbm.at[:, pl.ds(off, 128)], blk, sem)

    def process(b, blk):
        g = (b - lo_b) // (_NB // _NW // _NG + 1)

        def do_q(t, _):
            f0 = (g * _GQ + t) * _L
            mv = slc_v[f0 >> 7, pl.ds(f0 & 127, _L)]
            sv = sls_v[f0 >> 7, pl.ds(f0 & 127, _L)]
            mb = (mv >> 7) == b
            cnt = jnp.cumsum(jnp.where(mb, 1, 0))

            def do_match(tt, _):
                lane = jnp.sum(jnp.where(cnt <= tt, 1, 0))
                onlane = lanes == lane
                c = jnp.sum(jnp.where(onlane, mv, 0)) & 127
                slot = jnp.sum(jnp.where(onlane, sv, 0))
                cvec = jnp.full((_L,), c, jnp.int32)
                svec = jnp.full((_L,), slot, jnp.int32)
                for fb in range(2):
                    fvec = fb * _L + lanes
                    plsc.store_scatter(
                        pk_v, [fvec, svec],
                        plsc.load_gather(blk, [fvec, cvec]))
                for fb in range(2, 4):
                    fvec = fb * _L + lanes
                    plsc.store_scatter(
                        pk_v, [fvec, svec],
                        jnp.exp(plsc.load_gather(blk, [fvec, cvec])))
                return _

            lax.fori_loop(0, cnt[_L - 1], do_match, None)
            return _

        lax.fori_loop(0, _GQ, do_q, None)

    nb = hi_b - lo_b
    bufs = ((blk0_v, sem0), (blk1_v, sem1), (blk2_v, sem2),
            (blk3_v, sem3), (blk4_v, sem4))
    ndeep = len(bufs)
    for k in range(ndeep):
        fire(k, *bufs[k])

    def step(bi, _):
        for k in range(ndeep):
            @pl.when(bi % ndeep == k)
            def _k(k=k):
                blk, sem = bufs[k]
                pltpu.make_async_copy(
                    tabt_hbm.at[:, pl.ds(0, 128)], blk, sem).wait()
                process(lo_b + bi, blk)

                @pl.when(bi + ndeep < nb)
                def _f():
                    fire(bi + ndeep, blk, sem)
        return _

    lax.fori_loop(0, nb, step, None)

    base = pl.multiple_of(wid * _CAP, 128)
    pltpu.sync_copy(pk_v, packed_hbm.at[:, pl.ds(base, _CAP)])
    pltpu.sync_copy(mlp_v, pos_hbm.at[wid])


def _phase2_body(packed_hbm, pos_hbm, out_hbm, pk_v, pos_v, rows_v, sem):
    wid = lax.axis_index("s") * _NC + lax.axis_index("c")
    base = wid * _CAP
    lanes = lax.iota(jnp.int32, _L)

    pltpu.sync_copy(packed_hbm.at[:, pl.ds(base, _CAP)], pk_v)
    pltpu.sync_copy(pos_hbm.at[wid], pos_v)

    copies = []
    for ch in range(_CAP // 128):
        def tr(g, _, ch=ch):
            s0 = ch * 128 + g * _L
            rvec = g * _L + lanes
            for f in range(64):
                plsc.store_scatter(
                    rows_v, [rvec + ch * 128, jnp.full((_L,), f, jnp.int32)],
                    pk_v[f, pl.ds(s0, _L)])
            return _

        lax.fori_loop(0, 128 // _L, tr, None)
        copies.append(pltpu.async_copy(
            rows_v.at[pl.ds(ch * 128, 128), :],
            out_hbm.at[pos_v.at[ch]], sem))
    for c in copies:
        c.wait()


@jax.jit
def _prior_sc(indices, table_t):
    f32, i32 = jnp.float32, jnp.int32
    phase1 = functools.partial(
        pl.kernel,
        out_type=(jax.ShapeDtypeStruct((64, _NW * _CAP), f32),
                  jax.ShapeDtypeStruct((_NW, _CAP // 128, 128), i32)),
        mesh=plsc.VectorSubcoreMesh(core_axis_name="c", subcore_axis_name="s"),
        compiler_params=pltpu.CompilerParams(needs_layout_passes=False),
        scratch_types=[
            pltpu.VMEM((BATCH,), i32),
            pltpu.VMEM((_NQ // 8, 8 * _L), i32),
            pltpu.VMEM((_CAP // 128, 128), i32),
            pltpu.VMEM((_NG * _GCAP // 128, 128), i32),
            pltpu.VMEM((_NG * _GCAP // 128, 128), i32),
            pltpu.VMEM((64, _CAP), f32),
            pltpu.VMEM((64, 128), f32),
            pltpu.VMEM((64, 128), f32),
            pltpu.VMEM((64, 128), f32),
            pltpu.VMEM((64, 128), f32),
            pltpu.VMEM((64, 128), f32),
            pltpu.SemaphoreType.DMA,
            pltpu.SemaphoreType.DMA,
            pltpu.SemaphoreType.DMA,
            pltpu.SemaphoreType.DMA,
            pltpu.SemaphoreType.DMA,
        ],
    )(_phase1_body)
    packed, poslist = phase1(indices, table_t)

    phase2 = functools.partial(
        pl.kernel,
        out_type=jax.ShapeDtypeStruct((_OUT_ROWS, 64), f32),
        mesh=plsc.VectorSubcoreMesh(core_axis_name="c", subcore_axis_name="s"),
        compiler_params=pltpu.CompilerParams(use_tc_tiling_on_sc=False,
                                             needs_layout_passes=False),
        scratch_types=[
            pltpu.VMEM((64, _CAP), f32),
            pltpu.VMEM((_CAP // 128, 128), i32),
            pltpu.VMEM((_CAP, 64), f32),
            pltpu.SemaphoreType.DMA,
        ],
    )(_phase2_body)
    return phase2(packed, poslist)


def kernel(indices, table):
    out = _prior_sc(indices.astype(jnp.int32), table.T)
    return (out[:BATCH, :LAT_DIM], out[:BATCH, LAT_DIM:])
